# Initial kernel scaffold; baseline (speedup 1.0000x reference)
import jax
import jax.numpy as jnp
from jax.experimental import pallas as pl


def kernel(x):
    raise NotImplementedError("write your pallas kernel here")


# speedup vs baseline: 32.9104x; 32.9104x over previous
import jax
import jax.numpy as jnp
from jax.experimental import pallas as pl
from jax.experimental.pallas import tpu as pltpu

_ROWS, _COLS = 64, 32768
_N = _ROWS * _COLS
_PCT = 0.99999
_K_MIN = int(_N * (1 - _PCT))
_K_MAX = int(_N * _PCT)

def _to_sortable(x):
    top = jnp.uint32(0x80000000)
    bits = jax.lax.bitcast_convert_type(x, jnp.uint32)
    return jnp.where(bits >= top, ~bits, bits | top)


def _from_sortable(u):
    top = jnp.uint32(0x80000000)
    bits = jnp.where(u >= top, u ^ top, ~u)
    return jax.lax.bitcast_convert_type(bits, jnp.float32)


def _select_kernel(x_ref, max_ref, min_ref, u_ref):
    u_ref[...] = _to_sortable(x_ref[...])

    def body(i, carry):
        p_min, p_max = carry
        bit = jnp.left_shift(jnp.uint32(1), jnp.uint32(31) - i.astype(jnp.uint32))
        c_min = p_min | bit
        c_max = p_max | bit
        u = u_ref[...]
        cnt_min = jnp.sum((u < c_min).astype(jnp.int32))
        cnt_max = jnp.sum((u < c_max).astype(jnp.int32))
        p_min = jnp.where(cnt_min >= _K_MIN, p_min, c_min)
        p_max = jnp.where(cnt_max >= _K_MAX, p_max, c_max)
        return p_min, p_max

    p_min, p_max = jax.lax.fori_loop(
        0, 32, body, (jnp.uint32(0), jnp.uint32(0))
    )
    min_ref[0, 0] = _from_sortable(p_min)
    max_ref[0, 0] = _from_sortable(p_max)


def kernel(x):
    out_max, out_min = pl.pallas_call(
        _select_kernel,
        out_shape=(
            jax.ShapeDtypeStruct((1, 1), jnp.float32),
            jax.ShapeDtypeStruct((1, 1), jnp.float32),
        ),
        out_specs=(
            pl.BlockSpec(memory_space=pltpu.SMEM),
            pl.BlockSpec(memory_space=pltpu.SMEM),
        ),
        scratch_shapes=[pltpu.VMEM((_ROWS, _COLS), jnp.uint32)],
    )(x)
    return (out_max[0, 0], out_min[0, 0])
